# baseline (speedup 1.0000x reference)
import functools

import jax
import jax.numpy as jnp
from jax.experimental import pallas as pl
from jax.experimental.pallas import tpu as pltpu

_D = 2048
_HD = 128
_NH = 16
_NKV = 4
_REP = _NH // _NKV
_H = 7168
_V = 32000
_B = 4
_L = 512
_M = _B * _L
_EPS = 1e-5
_THETA = 10000.0

_TMQ = 256
_TMF = 1024
_TH = 256
_TV = 256
_GW = _REP * _HD


def _rmsnorm(x, nw):
    var = jnp.mean(x * x, axis=-1, keepdims=True)
    return (x * jax.lax.rsqrt(var + _EPS)) * nw


def _rope(y, cos, sin2):
    even = jax.lax.broadcasted_iota(jnp.int32, y.shape, 1) % 2 == 0
    n = y.shape[1]
    swap = jnp.where(even, pltpu.roll(y, n - 1, 1), pltpu.roll(y, 1, 1))
    return y * cos + swap * sin2


def _qkv_kernel(x_ref, nw_ref, wq_ref, wk_ref, wv_ref, q_ref, k_ref, v_ref):
    xn = _rmsnorm(x_ref[...], nw_ref[...])
    q_ref[...] = jnp.dot(xn, wq_ref[...], preferred_element_type=jnp.float32)
    k_ref[...] = jnp.dot(xn, wk_ref[...], preferred_element_type=jnp.float32)
    v_ref[...] = jnp.dot(xn, wv_ref[...], preferred_element_type=jnp.float32)


def _qkv_kernel_d(x_ref, d_ref, nw_ref, wq_ref, wk_ref, wv_ref,
                  q_ref, k_ref, v_ref):
    xn = _rmsnorm(x_ref[...] + d_ref[...], nw_ref[...])
    q_ref[...] = jnp.dot(xn, wq_ref[...], preferred_element_type=jnp.float32)
    k_ref[...] = jnp.dot(xn, wk_ref[...], preferred_element_type=jnp.float32)
    v_ref[...] = jnp.dot(xn, wv_ref[...], preferred_element_type=jnp.float32)


def _qkv(x, nw, wq, wk, wv, d=None):
    kvw = _NKV * _HD
    row_spec = pl.BlockSpec((_TMQ, _D), lambda i: (i, 0))
    w_specs = [
        pl.BlockSpec((1, _D), lambda i: (0, 0)),
        pl.BlockSpec((_D, _NH * _HD), lambda i: (0, 0)),
        pl.BlockSpec((_D, kvw), lambda i: (0, 0)),
        pl.BlockSpec((_D, kvw), lambda i: (0, 0)),
    ]
    if d is None:
        body, in_specs, args = _qkv_kernel, [row_spec], (x,)
    else:
        body, in_specs, args = _qkv_kernel_d, [row_spec, row_spec], (x, d)
    return pl.pallas_call(
        body,
        grid=(_M // _TMQ,),
        in_specs=in_specs + w_specs,
        out_specs=[
            pl.BlockSpec((_TMQ, _NH * _HD), lambda i: (i, 0)),
            pl.BlockSpec((_TMQ, kvw), lambda i: (i, 0)),
            pl.BlockSpec((_TMQ, kvw), lambda i: (i, 0)),
        ],
        out_shape=[
            jax.ShapeDtypeStruct((_M, _NH * _HD), jnp.float32),
            jax.ShapeDtypeStruct((_M, kvw), jnp.float32),
            jax.ShapeDtypeStruct((_M, kvw), jnp.float32),
        ],
        compiler_params=pltpu.CompilerParams(
            dimension_semantics=("parallel",),
            vmem_limit_bytes=56 * 1024 * 1024),
    )(*args, nw.reshape(1, _D), wq, wk, wv)


def _attn_core(scale, q_ref, k_ref, v_ref, cq_ref, sq_ref, ck_ref, sk_ref,
               wo_ref, fnw_ref, o_ref, on_ref, acc_ref, kv):
    q = _rope(q_ref[...], cq_ref[...], sq_ref[...])
    k = _rope(k_ref[...], ck_ref[...], sk_ref[...])
    v = v_ref[...]
    outs = []
    for r in range(_REP):
        qr = q[:, r * _HD:(r + 1) * _HD]
        s = jax.lax.dot_general(qr, k, (((1,), (1,)), ((), ())),
                                preferred_element_type=jnp.float32) * scale
        m = jnp.max(s, axis=-1, keepdims=True)
        p = jnp.exp(s - m)
        l = jnp.sum(p, axis=-1, keepdims=True)
        outs.append(jnp.dot(p, v, preferred_element_type=jnp.float32) / l)
    o = jnp.concatenate(outs, axis=1)
    acc_ref[...] += jnp.dot(o, wo_ref[...],
                            preferred_element_type=jnp.float32)

    @pl.when(kv == pl.num_programs(1) - 1)
    def _():
        h = acc_ref[...]
        o_ref[...] = h
        on_ref[...] = _rmsnorm(h, fnw_ref[...]).astype(jnp.bfloat16)


def _attn_wo_kernel(scale, q_ref, k_ref, v_ref, cq_ref, sq_ref, ck_ref,
                    sk_ref, wo_ref, fnw_ref, r_ref, o_ref, on_ref, acc_ref):
    kv = pl.program_id(1)

    @pl.when(kv == 0)
    def _():
        acc_ref[...] = r_ref[...]

    _attn_core(scale, q_ref, k_ref, v_ref, cq_ref, sq_ref, ck_ref, sk_ref,
               wo_ref, fnw_ref, o_ref, on_ref, acc_ref, kv)


def _attn_wo_kernel_d(scale, q_ref, k_ref, v_ref, cq_ref, sq_ref, ck_ref,
                      sk_ref, wo_ref, fnw_ref, r_ref, d_ref, o_ref, on_ref,
                      acc_ref):
    kv = pl.program_id(1)

    @pl.when(kv == 0)
    def _():
        acc_ref[...] = r_ref[...] + d_ref[...]

    _attn_core(scale, q_ref, k_ref, v_ref, cq_ref, sq_ref, ck_ref, sk_ref,
               wo_ref, fnw_ref, o_ref, on_ref, acc_ref, kv)


def _attn_wo(q, k, v, wo, ffn_nw, res, cos_t, sin_t, d=None):
    scale = float(_HD) ** -0.5
    res_spec = pl.BlockSpec((_L, _D), lambda b, kv: (b, 0))
    in_specs = [
        pl.BlockSpec((_L, _GW), lambda b, kv: (b, kv)),
        pl.BlockSpec((_L, _HD), lambda b, kv: (b, kv)),
        pl.BlockSpec((_L, _HD), lambda b, kv: (b, kv)),
        pl.BlockSpec((_L, _GW), lambda b, kv: (b, 0)),
        pl.BlockSpec((_L, _GW), lambda b, kv: (b, 0)),
        pl.BlockSpec((_L, _HD), lambda b, kv: (b, 0)),
        pl.BlockSpec((_L, _HD), lambda b, kv: (b, 0)),
        pl.BlockSpec((_GW, _D), lambda b, kv: (kv, 0)),
        pl.BlockSpec((1, _D), lambda b, kv: (0, 0)),
        res_spec,
    ]
    if d is None:
        body, args = _attn_wo_kernel, (res,)
    else:
        body, args = _attn_wo_kernel_d, (res, d)
        in_specs = in_specs + [res_spec]
    return pl.pallas_call(
        functools.partial(body, scale),
        grid=(_B, _NKV),
        in_specs=in_specs,
        out_specs=[pl.BlockSpec((_L, _D), lambda b, kv: (b, 0)),
                   pl.BlockSpec((_L, _D), lambda b, kv: (b, 0))],
        out_shape=[jax.ShapeDtypeStruct((_M, _D), jnp.float32),
                   jax.ShapeDtypeStruct((_M, _D), jnp.bfloat16)],
        scratch_shapes=[pltpu.VMEM((_L, _D), jnp.float32)],
        compiler_params=pltpu.CompilerParams(
            dimension_semantics=("parallel", "arbitrary"),
            vmem_limit_bytes=56 * 1024 * 1024),
    )(q, k, v, cos_t, sin_t, cos_t, sin_t, wo, ffn_nw.reshape(1, _D), *args)


def _ffn_kernel(xn_ref, w1_ref, w3_ref, w2_ref, o_ref):
    h = pl.program_id(1)
    xn = xn_ref[...]
    a = jnp.dot(xn, w1_ref[...].astype(jnp.bfloat16),
                preferred_element_type=jnp.float32)
    b = jnp.dot(xn, w3_ref[...].astype(jnp.bfloat16),
                preferred_element_type=jnp.float32)
    g = (a * jax.lax.logistic(a)) * b
    contrib = jnp.dot(g, w2_ref[...], preferred_element_type=jnp.float32)

    @pl.when(h == 0)
    def _():
        o_ref[...] = contrib

    @pl.when(h > 0)
    def _():
        o_ref[...] += contrib


def _ffn(xn, w1, w3, w2):
    return pl.pallas_call(
        _ffn_kernel,
        grid=(_M // _TMF, _H // _TH),
        in_specs=[
            pl.BlockSpec((_TMF, _D), lambda i, h: (i, 0)),
            pl.BlockSpec((_D, _TH), lambda i, h: (0, h)),
            pl.BlockSpec((_D, _TH), lambda i, h: (0, h)),
            pl.BlockSpec((_TH, _D), lambda i, h: (h, 0)),
        ],
        out_specs=pl.BlockSpec((_TMF, _D), lambda i, h: (i, 0)),
        out_shape=jax.ShapeDtypeStruct((_M, _D), jnp.float32),
        compiler_params=pltpu.CompilerParams(
            dimension_semantics=("parallel", "arbitrary"),
            vmem_limit_bytes=56 * 1024 * 1024),
    )(xn, w1, w3, w2)


def _norm_kernel(x_ref, d_ref, nw_ref, o_ref):
    o_ref[...] = _rmsnorm(x_ref[...] + d_ref[...],
                          nw_ref[...]).astype(jnp.bfloat16)


def _final_norm(x, d, nw):
    return pl.pallas_call(
        _norm_kernel,
        grid=(_M // _TMQ,),
        in_specs=[pl.BlockSpec((_TMQ, _D), lambda i: (i, 0)),
                  pl.BlockSpec((_TMQ, _D), lambda i: (i, 0)),
                  pl.BlockSpec((1, _D), lambda i: (0, 0))],
        out_specs=pl.BlockSpec((_TMQ, _D), lambda i: (i, 0)),
        out_shape=jax.ShapeDtypeStruct((_M, _D), jnp.bfloat16),
        compiler_params=pltpu.CompilerParams(
            dimension_semantics=("parallel",)),
    )(x, d, nw.reshape(1, _D))


def _logits_kernel(xn_ref, w_ref, o_ref):
    o_ref[...] = jnp.dot(xn_ref[...], w_ref[...].astype(jnp.bfloat16),
                         preferred_element_type=jnp.float32)


def _logits(xn, w_out):
    return pl.pallas_call(
        _logits_kernel,
        grid=(_V // _TV,),
        in_specs=[pl.BlockSpec((_M, _D), lambda j: (0, 0)),
                  pl.BlockSpec((_D, _TV), lambda j: (0, j))],
        out_specs=pl.BlockSpec((_M, _TV), lambda j: (0, j)),
        out_shape=jax.ShapeDtypeStruct((_M, _V), jnp.float32),
        compiler_params=pltpu.CompilerParams(
            dimension_semantics=("arbitrary",),
            vmem_limit_bytes=56 * 1024 * 1024),
    )(xn, w_out)


def _rope_tables():
    inv_freq = 1.0 / (_THETA ** (jnp.arange(0, _HD, 2, dtype=jnp.float32) / _HD))
    t = jnp.arange(_L, dtype=jnp.float32)
    freqs = t[:, None] * inv_freq[None, :]
    emb = jnp.concatenate([freqs, freqs], axis=-1)
    cos = jnp.cos(emb)
    sin = jnp.sin(emb)
    sign = jnp.where(jnp.arange(_HD) % 2 == 0, -1.0, 1.0)
    sin2 = sin * sign[None, :]
    cos_t = jnp.tile(cos, (_B, _REP))
    sin_t = jnp.tile(sin2, (_B, _REP))
    return cos_t, sin_t


def kernel(tokens, tok_emb, norm_w, w_out,
           l0_attn_norm, l0_ffn_norm, l0_wq, l0_wk, l0_wv, l0_wo,
           l0_w1, l0_w3, l0_w2,
           l1_attn_norm, l1_ffn_norm, l1_wq, l1_wk, l1_wv, l1_wo,
           l1_w1, l1_w3, l1_w2):
    cos_t, sin_t = _rope_tables()
    x = tok_emb[tokens].reshape(_M, _D)
    q, k, v = _qkv(x, l0_attn_norm, l0_wq, l0_wk, l0_wv)
    h0, hn0 = _attn_wo(q, k, v, l0_wo, l0_ffn_norm, x, cos_t, sin_t)
    d0 = _ffn(hn0, l0_w1, l0_w3, l0_w2)
    q, k, v = _qkv(h0, l1_attn_norm, l1_wq, l1_wk, l1_wv, d=d0)
    h1, hn1 = _attn_wo(q, k, v, l1_wo, l1_ffn_norm, h0, cos_t, sin_t, d=d0)
    d1 = _ffn(hn1, l1_w1, l1_w3, l1_w2)
    xn = _final_norm(h1, d1, norm_w)
    logits = _logits(xn, w_out)
    return logits.reshape(_B, _L, _V)
